# baseline (speedup 1.0000x reference)
import functools

import jax
import jax.numpy as jnp
from jax import lax
from jax.experimental import pallas as pl
from jax.experimental.pallas import tpu as pltpu
from jax.experimental.pallas import tpu_sc as plsc

VOCAB = 1000000
EMBED = 64
BATCH = 4096
SEQ = 50

NC = 2
NS = 16
NW = NC * NS
ROWS_PER_W = BATCH // NW
TOK_PER_W = ROWS_PER_W * SEQ
CHUNK = 128
NCHUNK = TOK_PER_W // CHUNK
PHYS = 2 * EMBED
RPB = 4000


TB = 1024
NB = 500
HALF = NB * TB
LASTB = (VOCAB - 1) // TB


def _repack_tc(tabT):
    def body(a_ref, b_ref, eye_ref, out_ref):
        tdims = (((0,), (0,)), ((), ()))
        out_ref[:, :EMBED] = lax.dot_general(
            a_ref[...], eye_ref[...], tdims,
            precision=lax.Precision.HIGHEST,
            preferred_element_type=jnp.float32)
        out_ref[:, EMBED:] = lax.dot_general(
            b_ref[...], eye_ref[...], tdims,
            precision=lax.Precision.HIGHEST,
            preferred_element_type=jnp.float32)

    return pl.pallas_call(
        body,
        grid=(NB,),
        in_specs=[
            pl.BlockSpec((EMBED, TB), lambda i: (0, jnp.minimum(2 * i, LASTB))),
            pl.BlockSpec((EMBED, TB),
                         lambda i: (0, jnp.minimum(2 * i + 1, LASTB))),
            pl.BlockSpec((EMBED, EMBED), lambda i: (0, 0)),
        ],
        out_specs=pl.BlockSpec((TB, PHYS), lambda i: (i, 0)),
        out_shape=jax.ShapeDtypeStruct((NB * TB, PHYS), jnp.float32),
    )(tabT, tabT, jnp.eye(EMBED, dtype=jnp.float32))


def _pooled_sum_sc(ids3, pattern3, zeros, tab2):
    mesh = plsc.VectorSubcoreMesh(core_axis_name="c", subcore_axis_name="s")

    @functools.partial(
        pl.kernel,
        out_type=jax.ShapeDtypeStruct((2 * BATCH, PHYS), jnp.float32),
        mesh=mesh,
        scratch_types=[
            pltpu.VMEM((NCHUNK, CHUNK), jnp.int32),
            pltpu.VMEM((NCHUNK, CHUNK), jnp.int32),
            pltpu.VMEM_SHARED((2 * NS * ROWS_PER_W, PHYS), jnp.float32),
            pltpu.VMEM((CHUNK, PHYS), jnp.float32),
            pltpu.VMEM((CHUNK, PHYS), jnp.float32),
            pltpu.SemaphoreType.DMA,
            pltpu.SemaphoreType.DMA,
        ],
    )
    def k(ids_hbm, pat_hbm, zer_hbm, tab_hbm, out_hbm,
          ids_v, pat_v, acc_sh, buf0, buf1, sem0, sem1):
        sid = lax.axis_index("s")
        wid = sid * NC + lax.axis_index("c")
        pltpu.sync_copy(ids_hbm.at[wid], ids_v)
        pltpu.sync_copy(pat_hbm.at[wid], pat_v)
        pltpu.sync_copy(zer_hbm, acc_sh.at[pl.ds(sid * 2 * ROWS_PER_W,
                                                 2 * ROWS_PER_W)])

        @pl.loop(0, NCHUNK, step=2)
        def _(j):
            c0 = pltpu.async_copy(tab_hbm.at[ids_v.at[j]], buf0, sem0)
            c1 = pltpu.async_copy(tab_hbm.at[ids_v.at[j + 1]], buf1, sem1)
            c0.wait()
            pltpu.sync_copy(buf0, acc_sh.at[pat_v.at[j]], add=True)
            c1.wait()
            pltpu.sync_copy(buf1, acc_sh.at[pat_v.at[j + 1]], add=True)

        pltpu.sync_copy(acc_sh.at[pl.ds(sid * 2 * ROWS_PER_W, 2 * ROWS_PER_W)],
                        out_hbm.at[pl.ds(wid * 2 * ROWS_PER_W, 2 * ROWS_PER_W)])

    return k(ids3, pattern3, zeros, tab2)


def _mlp_tc(acc, W1, b1, W2, b2):
    nblk = 8
    blk = BATCH // nblk

    def body(a_ref, w1_ref, b1_ref, w2_ref, b2_ref, o_ref):
        x = a_ref[:, 0, :EMBED] + a_ref[:, 1, EMBED:]
        x = x * (1.0 / SEQ)
        h = lax.dot_general(x, w1_ref[...], (((1,), (1,)), ((), ())),
                            precision=lax.Precision.HIGHEST,
                            preferred_element_type=jnp.float32)
        h = h + b1_ref[...]
        h = h * jax.nn.sigmoid(h)
        o = lax.dot_general(h, w2_ref[...], (((1,), (1,)), ((), ())),
                            precision=lax.Precision.HIGHEST,
                            preferred_element_type=jnp.float32)
        o_ref[...] = o + b2_ref[...]

    return pl.pallas_call(
        body,
        grid=(nblk,),
        in_specs=[
            pl.BlockSpec((blk, 2, PHYS), lambda i: (i, 0, 0)),
            pl.BlockSpec((EMBED, EMBED), lambda i: (0, 0)),
            pl.BlockSpec((1, EMBED), lambda i: (0, 0)),
            pl.BlockSpec((EMBED, EMBED), lambda i: (0, 0)),
            pl.BlockSpec((1, EMBED), lambda i: (0, 0)),
        ],
        out_specs=pl.BlockSpec((blk, EMBED), lambda i: (i, 0)),
        out_shape=jax.ShapeDtypeStruct((BATCH, EMBED), jnp.float32),
    )(acc, W1, b1.reshape(1, EMBED), W2, b2.reshape(1, EMBED))


def kernel(token_ids, table, W1, b1, W2, b2):
    tok3 = token_ids.reshape(NW, NCHUNK, CHUNK).astype(jnp.int32)
    c2 = tok3 // TB
    half = c2 & 1
    ids3 = (c2 >> 1) * TB + (tok3 % TB)
    sid3 = (jnp.arange(NW, dtype=jnp.int32) // NC).reshape(NW, 1, 1)
    row3 = (jnp.arange(TOK_PER_W, dtype=jnp.int32) // SEQ).reshape(1, NCHUNK, CHUNK)
    pattern3 = 2 * (sid3 * ROWS_PER_W + row3) + half
    zeros = jnp.zeros((2 * ROWS_PER_W, PHYS), jnp.float32)
    tab2 = _repack_tc(table.T)
    acc = _pooled_sum_sc(ids3, pattern3, zeros, tab2)
    return _mlp_tc(acc.reshape(BATCH, 2, PHYS), W1, b1, W2, b2)


# speedup vs baseline: 1.3162x; 1.3162x over previous
import functools

import jax
import jax.numpy as jnp
from jax import lax
from jax.experimental import pallas as pl
from jax.experimental.pallas import tpu as pltpu
from jax.experimental.pallas import tpu_sc as plsc

VOCAB = 1000000
EMBED = 64
BATCH = 4096
SEQ = 50

NC = 2
NS = 16
NW = NC * NS
ROWS_PER_W = BATCH // NW
TOK_PER_W = ROWS_PER_W * SEQ
CHUNK = 128
NCHUNK = TOK_PER_W // CHUNK
PHYS = 2 * EMBED
RPB = 4000


TB = 1024
NB = 500
HALF = NB * TB
LASTB = (VOCAB - 1) // TB


def _repack_tc(tabT):
    def body(a_ref, b_ref, out_ref):
        out_ref[:, :EMBED] = a_ref[...].T
        out_ref[:, EMBED:] = b_ref[...].T

    return pl.pallas_call(
        body,
        grid=(NB,),
        in_specs=[
            pl.BlockSpec((EMBED, TB), lambda i: (0, jnp.minimum(2 * i, LASTB))),
            pl.BlockSpec((EMBED, TB),
                         lambda i: (0, jnp.minimum(2 * i + 1, LASTB))),
        ],
        out_specs=pl.BlockSpec((TB, PHYS), lambda i: (i, 0)),
        out_shape=jax.ShapeDtypeStruct((NB * TB, PHYS), jnp.float32),
        compiler_params=pltpu.CompilerParams(
            dimension_semantics=("parallel",)),
    )(tabT, tabT)


def _pooled_sum_sc(ids3, pattern3, zeros, tab2):
    mesh = plsc.VectorSubcoreMesh(core_axis_name="c", subcore_axis_name="s")

    @functools.partial(
        pl.kernel,
        out_type=jax.ShapeDtypeStruct((2 * BATCH, PHYS), jnp.float32),
        mesh=mesh,
        scratch_types=[
            pltpu.VMEM((NCHUNK, CHUNK), jnp.int32),
            pltpu.VMEM((NCHUNK, CHUNK), jnp.int32),
            pltpu.VMEM_SHARED((2 * NS * ROWS_PER_W, PHYS), jnp.float32),
            pltpu.VMEM((CHUNK, PHYS), jnp.float32),
            pltpu.VMEM((CHUNK, PHYS), jnp.float32),
            pltpu.SemaphoreType.DMA,
            pltpu.SemaphoreType.DMA,
        ],
    )
    def k(ids_hbm, pat_hbm, zer_hbm, tab_hbm, out_hbm,
          ids_v, pat_v, acc_sh, buf0, buf1, sem0, sem1):
        sid = lax.axis_index("s")
        wid = sid * NC + lax.axis_index("c")
        pltpu.sync_copy(ids_hbm.at[wid], ids_v)
        pltpu.sync_copy(pat_hbm.at[wid], pat_v)
        pltpu.sync_copy(zer_hbm, acc_sh.at[pl.ds(sid * 2 * ROWS_PER_W,
                                                 2 * ROWS_PER_W)])

        @pl.loop(0, NCHUNK, step=2)
        def _(j):
            c0 = pltpu.async_copy(tab_hbm.at[ids_v.at[j]], buf0, sem0)
            c1 = pltpu.async_copy(tab_hbm.at[ids_v.at[j + 1]], buf1, sem1)
            c0.wait()
            pltpu.sync_copy(buf0, acc_sh.at[pat_v.at[j]], add=True)
            c1.wait()
            pltpu.sync_copy(buf1, acc_sh.at[pat_v.at[j + 1]], add=True)

        pltpu.sync_copy(acc_sh.at[pl.ds(sid * 2 * ROWS_PER_W, 2 * ROWS_PER_W)],
                        out_hbm.at[pl.ds(wid * 2 * ROWS_PER_W, 2 * ROWS_PER_W)])

    return k(ids3, pattern3, zeros, tab2)


def _mlp_tc(acc, W1, b1, W2, b2):
    nblk = 8
    blk = BATCH // nblk

    def body(a_ref, w1_ref, b1_ref, w2_ref, b2_ref, o_ref):
        x = a_ref[:, 0, :EMBED] + a_ref[:, 1, EMBED:]
        x = x * (1.0 / SEQ)
        h = lax.dot_general(x, w1_ref[...], (((1,), (1,)), ((), ())),
                            precision=lax.Precision.HIGHEST,
                            preferred_element_type=jnp.float32)
        h = h + b1_ref[...]
        h = h * jax.nn.sigmoid(h)
        o = lax.dot_general(h, w2_ref[...], (((1,), (1,)), ((), ())),
                            precision=lax.Precision.HIGHEST,
                            preferred_element_type=jnp.float32)
        o_ref[...] = o + b2_ref[...]

    return pl.pallas_call(
        body,
        grid=(nblk,),
        in_specs=[
            pl.BlockSpec((blk, 2, PHYS), lambda i: (i, 0, 0)),
            pl.BlockSpec((EMBED, EMBED), lambda i: (0, 0)),
            pl.BlockSpec((1, EMBED), lambda i: (0, 0)),
            pl.BlockSpec((EMBED, EMBED), lambda i: (0, 0)),
            pl.BlockSpec((1, EMBED), lambda i: (0, 0)),
        ],
        out_specs=pl.BlockSpec((blk, EMBED), lambda i: (i, 0)),
        out_shape=jax.ShapeDtypeStruct((BATCH, EMBED), jnp.float32),
    )(acc, W1, b1.reshape(1, EMBED), W2, b2.reshape(1, EMBED))


def kernel(token_ids, table, W1, b1, W2, b2):
    tok3 = token_ids.reshape(NW, NCHUNK, CHUNK).astype(jnp.int32)
    c2 = tok3 // TB
    half = c2 & 1
    ids3 = (c2 >> 1) * TB + (tok3 % TB)
    sid3 = (jnp.arange(NW, dtype=jnp.int32) // NC).reshape(NW, 1, 1)
    row3 = (jnp.arange(TOK_PER_W, dtype=jnp.int32) // SEQ).reshape(1, NCHUNK, CHUNK)
    pattern3 = 2 * (sid3 * ROWS_PER_W + row3) + half
    zeros = jnp.zeros((2 * ROWS_PER_W, PHYS), jnp.float32)
    tab2 = _repack_tc(table.T)
    acc = _pooled_sum_sc(ids3, pattern3, zeros, tab2)
    return _mlp_tc(acc.reshape(BATCH, 2, PHYS), W1, b1, W2, b2)


# speedup vs baseline: 1.6667x; 1.2663x over previous
import functools

import jax
import jax.numpy as jnp
from jax import lax
from jax.experimental import pallas as pl
from jax.experimental.pallas import tpu as pltpu
from jax.experimental.pallas import tpu_sc as plsc

VOCAB = 1000000
EMBED = 64
BATCH = 4096
SEQ = 50

NC = 2
NS = 16
NW = NC * NS
ROWS_PER_W = BATCH // NW
TOK_PER_W = ROWS_PER_W * SEQ
CHUNK = 128
NCHUNK = TOK_PER_W // CHUNK
PHYS = 2 * EMBED
RPB = 4000


TB = 2048
NB = 250
HALF = NB * TB
LASTB = (VOCAB - 1) // TB


def _repack_tc(tabT):
    def body(a_ref, b_ref, out_ref):
        out_ref[:, :EMBED] = a_ref[...].T
        out_ref[:, EMBED:] = b_ref[...].T

    return pl.pallas_call(
        body,
        grid=(NB,),
        in_specs=[
            pl.BlockSpec((EMBED, TB), lambda i: (0, jnp.minimum(2 * i, LASTB))),
            pl.BlockSpec((EMBED, TB),
                         lambda i: (0, jnp.minimum(2 * i + 1, LASTB))),
        ],
        out_specs=pl.BlockSpec((TB, PHYS), lambda i: (i, 0)),
        out_shape=jax.ShapeDtypeStruct((NB * TB, PHYS), jnp.float32),
        compiler_params=pltpu.CompilerParams(
            dimension_semantics=("parallel",)),
    )(tabT, tabT)


def _pooled_sum_sc(ids3, pattern3, zeros, tab2):
    mesh = plsc.VectorSubcoreMesh(core_axis_name="c", subcore_axis_name="s")

    @functools.partial(
        pl.kernel,
        out_type=jax.ShapeDtypeStruct((2 * BATCH, PHYS), jnp.float32),
        mesh=mesh,
        scratch_types=[
            pltpu.VMEM((NCHUNK, CHUNK), jnp.int32),
            pltpu.VMEM((NCHUNK, CHUNK), jnp.int32),
            pltpu.VMEM_SHARED((2 * NS * ROWS_PER_W, PHYS), jnp.float32),
            pltpu.VMEM((CHUNK, PHYS), jnp.float32),
            pltpu.VMEM((CHUNK, PHYS), jnp.float32),
            pltpu.SemaphoreType.DMA,
            pltpu.SemaphoreType.DMA,
        ],
    )
    def k(ids_hbm, pat_hbm, zer_hbm, tab_hbm, out_hbm,
          ids_v, pat_v, acc_sh, buf0, buf1, sem0, sem1):
        sid = lax.axis_index("s")
        wid = sid * NC + lax.axis_index("c")
        pltpu.sync_copy(ids_hbm.at[wid], ids_v)
        pltpu.sync_copy(pat_hbm.at[wid], pat_v)
        pltpu.sync_copy(zer_hbm, acc_sh.at[pl.ds(sid * 2 * ROWS_PER_W,
                                                 2 * ROWS_PER_W)])

        @pl.loop(0, NCHUNK, step=2)
        def _(j):
            c0 = pltpu.async_copy(tab_hbm.at[ids_v.at[j]], buf0, sem0)
            c1 = pltpu.async_copy(tab_hbm.at[ids_v.at[j + 1]], buf1, sem1)
            c0.wait()
            pltpu.sync_copy(buf0, acc_sh.at[pat_v.at[j]], add=True)
            c1.wait()
            pltpu.sync_copy(buf1, acc_sh.at[pat_v.at[j + 1]], add=True)

        pltpu.sync_copy(acc_sh.at[pl.ds(sid * 2 * ROWS_PER_W, 2 * ROWS_PER_W)],
                        out_hbm.at[pl.ds(wid * 2 * ROWS_PER_W, 2 * ROWS_PER_W)])

    return k(ids3, pattern3, zeros, tab2)


def _mlp_tc(acc, W1, b1, W2, b2):
    nblk = 8
    blk = BATCH // nblk

    def body(a_ref, w1_ref, b1_ref, w2_ref, b2_ref, o_ref):
        x = a_ref[:, 0, :EMBED] + a_ref[:, 1, EMBED:]
        x = x * (1.0 / SEQ)
        h = lax.dot_general(x, w1_ref[...], (((1,), (1,)), ((), ())),
                            precision=lax.Precision.HIGHEST,
                            preferred_element_type=jnp.float32)
        h = h + b1_ref[...]
        h = h * jax.nn.sigmoid(h)
        o = lax.dot_general(h, w2_ref[...], (((1,), (1,)), ((), ())),
                            precision=lax.Precision.HIGHEST,
                            preferred_element_type=jnp.float32)
        o_ref[...] = o + b2_ref[...]

    return pl.pallas_call(
        body,
        grid=(nblk,),
        in_specs=[
            pl.BlockSpec((blk, 2, PHYS), lambda i: (i, 0, 0)),
            pl.BlockSpec((EMBED, EMBED), lambda i: (0, 0)),
            pl.BlockSpec((1, EMBED), lambda i: (0, 0)),
            pl.BlockSpec((EMBED, EMBED), lambda i: (0, 0)),
            pl.BlockSpec((1, EMBED), lambda i: (0, 0)),
        ],
        out_specs=pl.BlockSpec((blk, EMBED), lambda i: (i, 0)),
        out_shape=jax.ShapeDtypeStruct((BATCH, EMBED), jnp.float32),
    )(acc, W1, b1.reshape(1, EMBED), W2, b2.reshape(1, EMBED))


def kernel(token_ids, table, W1, b1, W2, b2):
    tok3 = token_ids.reshape(NW, NCHUNK, CHUNK).astype(jnp.int32)
    c2 = tok3 // TB
    half = c2 & 1
    ids3 = (c2 >> 1) * TB + (tok3 % TB)
    sid3 = (jnp.arange(NW, dtype=jnp.int32) // NC).reshape(NW, 1, 1)
    row3 = (jnp.arange(TOK_PER_W, dtype=jnp.int32) // SEQ).reshape(1, NCHUNK, CHUNK)
    pattern3 = 2 * (sid3 * ROWS_PER_W + row3) + half
    zeros = jnp.zeros((2 * ROWS_PER_W, PHYS), jnp.float32)
    tab2 = _repack_tc(table.T)
    acc = _pooled_sum_sc(ids3, pattern3, zeros, tab2)
    return _mlp_tc(acc.reshape(BATCH, 2, PHYS), W1, b1, W2, b2)


# speedup vs baseline: 1.9680x; 1.1808x over previous
import functools

import jax
import jax.numpy as jnp
from jax import lax
from jax.experimental import pallas as pl
from jax.experimental.pallas import tpu as pltpu
from jax.experimental.pallas import tpu_sc as plsc

VOCAB = 1000000
EMBED = 64
BATCH = 4096
SEQ = 50

NC = 2
NS = 16
NW = NC * NS
ROWS_PER_W = BATCH // NW
TOK_PER_W = ROWS_PER_W * SEQ
CHUNK = 128
NCHUNK = TOK_PER_W // CHUNK
PHYS = 2 * EMBED
RPB = 4000


TB = 4096
NB = 125
HALF = NB * TB
LASTB = (VOCAB - 1) // TB


def _repack_tc(tabT):
    def body(a_ref, b_ref, out_ref):
        out_ref[:, :EMBED] = a_ref[...].T
        out_ref[:, EMBED:] = b_ref[...].T

    return pl.pallas_call(
        body,
        grid=(NB,),
        in_specs=[
            pl.BlockSpec((EMBED, TB), lambda i: (0, jnp.minimum(2 * i, LASTB))),
            pl.BlockSpec((EMBED, TB),
                         lambda i: (0, jnp.minimum(2 * i + 1, LASTB))),
        ],
        out_specs=pl.BlockSpec((TB, PHYS), lambda i: (i, 0)),
        out_shape=jax.ShapeDtypeStruct((NB * TB, PHYS), jnp.float32),
        compiler_params=pltpu.CompilerParams(
            dimension_semantics=("parallel",)),
    )(tabT, tabT)


def _pooled_sum_sc(ids3, pattern3, zeros, tab2):
    mesh = plsc.VectorSubcoreMesh(core_axis_name="c", subcore_axis_name="s")

    @functools.partial(
        pl.kernel,
        out_type=jax.ShapeDtypeStruct((2 * BATCH, PHYS), jnp.float32),
        mesh=mesh,
        scratch_types=[
            pltpu.VMEM((NCHUNK, CHUNK), jnp.int32),
            pltpu.VMEM((NCHUNK, CHUNK), jnp.int32),
            pltpu.VMEM_SHARED((2 * NS * ROWS_PER_W, PHYS), jnp.float32),
            pltpu.VMEM((CHUNK, PHYS), jnp.float32),
            pltpu.VMEM((CHUNK, PHYS), jnp.float32),
            pltpu.SemaphoreType.DMA,
            pltpu.SemaphoreType.DMA,
        ],
    )
    def k(ids_hbm, pat_hbm, zer_hbm, tab_hbm, out_hbm,
          ids_v, pat_v, acc_sh, buf0, buf1, sem0, sem1):
        sid = lax.axis_index("s")
        wid = sid * NC + lax.axis_index("c")
        pltpu.sync_copy(ids_hbm.at[wid], ids_v)
        pltpu.sync_copy(pat_hbm.at[wid], pat_v)
        pltpu.sync_copy(zer_hbm, acc_sh.at[pl.ds(sid * 2 * ROWS_PER_W,
                                                 2 * ROWS_PER_W)])

        @pl.loop(0, NCHUNK, step=2)
        def _(j):
            c0 = pltpu.async_copy(tab_hbm.at[ids_v.at[j]], buf0, sem0)
            c1 = pltpu.async_copy(tab_hbm.at[ids_v.at[j + 1]], buf1, sem1)
            c0.wait()
            pltpu.sync_copy(buf0, acc_sh.at[pat_v.at[j]], add=True)
            c1.wait()
            pltpu.sync_copy(buf1, acc_sh.at[pat_v.at[j + 1]], add=True)

        pltpu.sync_copy(acc_sh.at[pl.ds(sid * 2 * ROWS_PER_W, 2 * ROWS_PER_W)],
                        out_hbm.at[pl.ds(wid * 2 * ROWS_PER_W, 2 * ROWS_PER_W)])

    return k(ids3, pattern3, zeros, tab2)


def _mlp_tc(acc, W1, b1, W2, b2):
    nblk = 8
    blk = BATCH // nblk

    def body(a_ref, w1_ref, b1_ref, w2_ref, b2_ref, o_ref):
        x = a_ref[:, 0, :EMBED] + a_ref[:, 1, EMBED:]
        x = x * (1.0 / SEQ)
        h = lax.dot_general(x, w1_ref[...], (((1,), (1,)), ((), ())),
                            precision=lax.Precision.HIGHEST,
                            preferred_element_type=jnp.float32)
        h = h + b1_ref[...]
        h = h * jax.nn.sigmoid(h)
        o = lax.dot_general(h, w2_ref[...], (((1,), (1,)), ((), ())),
                            precision=lax.Precision.HIGHEST,
                            preferred_element_type=jnp.float32)
        o_ref[...] = o + b2_ref[...]

    return pl.pallas_call(
        body,
        grid=(nblk,),
        in_specs=[
            pl.BlockSpec((blk, 2, PHYS), lambda i: (i, 0, 0)),
            pl.BlockSpec((EMBED, EMBED), lambda i: (0, 0)),
            pl.BlockSpec((1, EMBED), lambda i: (0, 0)),
            pl.BlockSpec((EMBED, EMBED), lambda i: (0, 0)),
            pl.BlockSpec((1, EMBED), lambda i: (0, 0)),
        ],
        out_specs=pl.BlockSpec((blk, EMBED), lambda i: (i, 0)),
        out_shape=jax.ShapeDtypeStruct((BATCH, EMBED), jnp.float32),
    )(acc, W1, b1.reshape(1, EMBED), W2, b2.reshape(1, EMBED))


def kernel(token_ids, table, W1, b1, W2, b2):
    tok3 = token_ids.reshape(NW, NCHUNK, CHUNK).astype(jnp.int32)
    c2 = tok3 // TB
    half = c2 & 1
    ids3 = (c2 >> 1) * TB + (tok3 % TB)
    sid3 = (jnp.arange(NW, dtype=jnp.int32) // NC).reshape(NW, 1, 1)
    row3 = (jnp.arange(TOK_PER_W, dtype=jnp.int32) // SEQ).reshape(1, NCHUNK, CHUNK)
    pattern3 = 2 * (sid3 * ROWS_PER_W + row3) + half
    zeros = jnp.zeros((2 * ROWS_PER_W, PHYS), jnp.float32)
    tab2 = _repack_tc(table.T)
    acc = _pooled_sum_sc(ids3, pattern3, zeros, tab2)
    return _mlp_tc(acc.reshape(BATCH, 2, PHYS), W1, b1, W2, b2)
